# Initial kernel scaffold; baseline (speedup 1.0000x reference)
import jax
import jax.numpy as jnp
from jax.experimental import pallas as pl


def kernel(x, Wr, w1, w2, w3, sw1, sw2, sw3):
    raise NotImplementedError("write your pallas kernel here")


# speedup vs baseline: 3.8366x; 3.8366x over previous
import functools
import jax
import jax.numpy as jnp
from jax.experimental import pallas as pl
from jax.experimental.pallas import tpu as pltpu

T = 4096
D = 1024
H = 1024
E = 8
K = 2
COEF = 0.01

B = 256
R = T * K + T
NB = R // B
G = E + 1
NT = NB + G - 1


def _gelu(v):
    return 0.5 * v * (1.0 + jax.lax.erf(v * 0.7071067811865476))


def _router_body(x_ref, wr_ref, out_ref):
    out_ref[...] = jax.lax.dot_general(
        x_ref[...], wr_ref[...], (((1,), (1,)), ((), ())),
        preferred_element_type=jnp.float32)


def _router(x, Wr):
    blk = 512
    return pl.pallas_call(
        _router_body,
        grid=(T // blk,),
        in_specs=[
            pl.BlockSpec((blk, D), lambda i: (i, 0)),
            pl.BlockSpec((E, D), lambda i: (0, 0)),
        ],
        out_specs=pl.BlockSpec((blk, E), lambda i: (i, 0)),
        out_shape=jax.ShapeDtypeStruct((T, E), jnp.float32),
    )(x, Wr)


def _ffn_body(grp_ref, blk_ref, lo_ref, hi_ref, first_ref,
              xs_ref, w1_ref, w3_ref, w2_ref, gate_ref, out_ref):
    i = pl.program_id(0)
    lo = lo_ref[i]
    hi = hi_ref[i]
    base = blk_ref[i] * B
    rows = base + jax.lax.broadcasted_iota(jnp.int32, (B, 1), 0)
    mask = (rows >= lo) & (rows < hi)

    x = xs_ref[...]
    h1 = jax.lax.dot_general(x, w1_ref[0], (((1,), (1,)), ((), ())),
                             preferred_element_type=jnp.float32)
    h3 = jax.lax.dot_general(x, w3_ref[0], (((1,), (1,)), ((), ())),
                             preferred_element_type=jnp.float32)
    h = _gelu(h1) * h3
    o = jax.lax.dot_general(h, w2_ref[0], (((1,), (1,)), ((), ())),
                            preferred_element_type=jnp.float32)
    o = o * gate_ref[...]
    o = jnp.where(mask, o, 0.0)

    @pl.when(first_ref[i] == 1)
    def _():
        out_ref[...] = o

    @pl.when(first_ref[i] == 0)
    def _():
        out_ref[...] += o


def _grouped_ffn(xs, w1s, w3s, w2s, gates, grp, blk, lo, hi, first):
    grid_spec = pltpu.PrefetchScalarGridSpec(
        num_scalar_prefetch=5,
        grid=(NT,),
        in_specs=[
            pl.BlockSpec((B, D), lambda i, g, b, lo, hi, f: (b[i], 0)),
            pl.BlockSpec((1, H, D), lambda i, g, b, lo, hi, f: (g[i], 0, 0)),
            pl.BlockSpec((1, H, D), lambda i, g, b, lo, hi, f: (g[i], 0, 0)),
            pl.BlockSpec((1, D, H), lambda i, g, b, lo, hi, f: (g[i], 0, 0)),
            pl.BlockSpec((B, 1), lambda i, g, b, lo, hi, f: (b[i], 0)),
        ],
        out_specs=pl.BlockSpec((B, D), lambda i, g, b, lo, hi, f: (b[i], 0)),
    )
    return pl.pallas_call(
        _ffn_body,
        grid_spec=grid_spec,
        out_shape=jax.ShapeDtypeStruct((R, D), jnp.float32),
    )(grp, blk, lo, hi, first, xs, w1s, w3s, w2s, gates)


def _tile_maps(offsets):
    s = offsets[:-1]
    t = offsets[1:]
    nonempty = t > s
    bs = s // B
    be = jnp.where(nonempty, (t + B - 1) // B, bs)
    tiles_per = jnp.where(nonempty, be - bs, 0)
    start = jnp.concatenate([jnp.zeros((1,), jnp.int32),
                             jnp.cumsum(tiles_per)]).astype(jnp.int32)
    total = start[-1]
    j = jnp.arange(NT, dtype=jnp.int32)
    g = jnp.searchsorted(start[1:], j, side="right").astype(jnp.int32)
    g = jnp.minimum(g, G - 1)
    blk = bs[g] + (j - start[g])
    lo = jnp.maximum(s[g], blk * B)
    hi = jnp.minimum(t[g], (blk + 1) * B)
    valid = j < total
    last_blk = blk[jnp.maximum(total - 1, 0)]
    blk = jnp.where(valid, blk, last_blk)
    lo = jnp.where(valid, lo, 0)
    hi = jnp.where(valid, hi, 0)
    prev_blk = jnp.concatenate([jnp.full((1,), -1, jnp.int32), blk[:-1]])
    first = (valid & (blk != prev_blk)).astype(jnp.int32)
    return g, blk.astype(jnp.int32), lo.astype(jnp.int32), hi.astype(jnp.int32), first


@jax.jit
def kernel(x, Wr, w1, w2, w3, sw1, sw2, sw3):
    logits = _router(x, Wr)

    m1 = jnp.max(logits, axis=-1)
    i1 = jnp.argmax(logits, axis=-1).astype(jnp.int32)
    masked = jnp.where(jax.nn.one_hot(i1, E, dtype=jnp.bool_), -jnp.inf, logits)
    i2 = jnp.argmax(masked, axis=-1).astype(jnp.int32)
    ex = jnp.exp(logits - m1[:, None])
    denom = jnp.sum(ex, axis=-1)
    probs = ex / denom[:, None]
    g1 = jnp.take_along_axis(probs, i1[:, None], axis=1)[:, 0]
    g2 = jnp.take_along_axis(probs, i2[:, None], axis=1)[:, 0]
    density = jnp.mean(jax.nn.one_hot(i1, E, dtype=jnp.float32), axis=0)
    prob_mean = jnp.mean(probs, axis=0)
    aux_loss = COEF * jnp.sum(density * prob_mean) * E

    flat_e = jnp.stack([i1, i2], axis=1).reshape(-1)
    flat_g = jnp.stack([g1, g2], axis=1).reshape(-1)

    perm = jnp.argsort(flat_e, stable=True).astype(jnp.int32)
    inv = jnp.argsort(perm, stable=True).astype(jnp.int32)
    counts = jnp.bincount(flat_e, length=E)
    offsets = jnp.concatenate([
        jnp.zeros((1,), jnp.int32),
        jnp.cumsum(counts).astype(jnp.int32),
        jnp.full((1,), R, jnp.int32),
    ])
    grp, blk, lo, hi, first = _tile_maps(offsets)

    ar = jnp.arange(T, dtype=jnp.int32)
    sorted_tok = jnp.concatenate([perm // K, ar])
    xs = x[sorted_tok]
    gates = jnp.concatenate([flat_g[perm], jnp.ones((T,), jnp.float32)])
    gates = gates[:, None]

    w1s = jnp.concatenate([w1, sw1[None]], axis=0)
    w3s = jnp.concatenate([w3, sw3[None]], axis=0)
    w2s = jnp.concatenate([w2, sw2[None]], axis=0)

    os = _grouped_ffn(xs, w1s, w3s, w2s, gates, grp, blk, lo, hi, first)

    p0 = inv[ar * K]
    p1 = inv[ar * K + 1]
    out = os[p0] + os[p1] + os[T * K + ar]
    return out, aux_loss


# speedup vs baseline: 3.9611x; 1.0325x over previous
import functools
import jax
import jax.numpy as jnp
from jax import lax
from jax.experimental import pallas as pl
from jax.experimental.pallas import tpu as pltpu
from jax.experimental.pallas import tpu_sc as plsc

T = 4096
D = 1024
H = 1024
E = 8
K = 2
COEF = 0.01

B = 256
R = T * K + T
NB = R // B
G = E + 1
NT = NB + G - 1


def _gelu(v):
    return 0.5 * v * (1.0 + jax.lax.erf(v * 0.7071067811865476))


def _router_body(x_ref, wr_ref, out_ref):
    out_ref[...] = jax.lax.dot_general(
        x_ref[...], wr_ref[...], (((1,), (1,)), ((), ())),
        preferred_element_type=jnp.float32)


def _router(x, Wr):
    blk = 512
    return pl.pallas_call(
        _router_body,
        grid=(T // blk,),
        in_specs=[
            pl.BlockSpec((blk, D), lambda i: (i, 0)),
            pl.BlockSpec((E, D), lambda i: (0, 0)),
        ],
        out_specs=pl.BlockSpec((blk, E), lambda i: (i, 0)),
        out_shape=jax.ShapeDtypeStruct((T, E), jnp.float32),
    )(x, Wr)


def _ffn_body(grp_ref, blk_ref, lo_ref, hi_ref, first_ref,
              xs_ref, w1_ref, w3_ref, w2_ref, out_ref):
    i = pl.program_id(0)
    lo = lo_ref[i]
    hi = hi_ref[i]
    base = blk_ref[i] * B
    rows = base + jax.lax.broadcasted_iota(jnp.int32, (B, 1), 0)
    mask = (rows >= lo) & (rows < hi)

    x = xs_ref[...]
    h1 = jax.lax.dot_general(x, w1_ref[0], (((1,), (1,)), ((), ())),
                             preferred_element_type=jnp.float32)
    h3 = jax.lax.dot_general(x, w3_ref[0], (((1,), (1,)), ((), ())),
                             preferred_element_type=jnp.float32)
    h = _gelu(h1) * h3
    o = jax.lax.dot_general(h, w2_ref[0], (((1,), (1,)), ((), ())),
                            preferred_element_type=jnp.float32)
    o = jnp.where(mask, o, 0.0)

    @pl.when(first_ref[i] == 1)
    def _():
        out_ref[...] = o

    @pl.when(first_ref[i] == 0)
    def _():
        out_ref[...] += o


def _grouped_ffn(xs, w1s, w3s, w2s, grp, blk, lo, hi, first):
    grid_spec = pltpu.PrefetchScalarGridSpec(
        num_scalar_prefetch=5,
        grid=(NT,),
        in_specs=[
            pl.BlockSpec((B, D), lambda i, g, b, lo, hi, f: (b[i], 0)),
            pl.BlockSpec((1, H, D), lambda i, g, b, lo, hi, f: (g[i], 0, 0)),
            pl.BlockSpec((1, H, D), lambda i, g, b, lo, hi, f: (g[i], 0, 0)),
            pl.BlockSpec((1, D, H), lambda i, g, b, lo, hi, f: (g[i], 0, 0)),
        ],
        out_specs=pl.BlockSpec((B, D), lambda i, g, b, lo, hi, f: (b[i], 0)),
    )
    return pl.pallas_call(
        _ffn_body,
        grid_spec=grid_spec,
        out_shape=jax.ShapeDtypeStruct((R, D), jnp.float32),
    )(grp, blk, lo, hi, first, xs, w1s, w3s, w2s)


def _tile_maps(offsets):
    s = offsets[:-1]
    t = offsets[1:]
    nonempty = t > s
    bs = s // B
    be = jnp.where(nonempty, (t + B - 1) // B, bs)
    tiles_per = jnp.where(nonempty, be - bs, 0)
    start = jnp.concatenate([jnp.zeros((1,), jnp.int32),
                             jnp.cumsum(tiles_per)]).astype(jnp.int32)
    total = start[-1]
    j = jnp.arange(NT, dtype=jnp.int32)
    g = jnp.searchsorted(start[1:], j, side="right").astype(jnp.int32)
    g = jnp.minimum(g, G - 1)
    blk = bs[g] + (j - start[g])
    lo = jnp.maximum(s[g], blk * B)
    hi = jnp.minimum(t[g], (blk + 1) * B)
    valid = j < total
    last_blk = blk[jnp.maximum(total - 1, 0)]
    blk = jnp.where(valid, blk, last_blk)
    lo = jnp.where(valid, lo, 0)
    hi = jnp.where(valid, hi, 0)
    prev_blk = jnp.concatenate([jnp.full((1,), -1, jnp.int32), blk[:-1]])
    first = (valid & (blk != prev_blk)).astype(jnp.int32)
    return g, blk.astype(jnp.int32), lo.astype(jnp.int32), hi.astype(jnp.int32), first


_NW = 32
_TPW = T // _NW
_CH = 16
_NCH = _TPW // _CH


def _combine_body(os_hbm, inv_hbm, g1_hbm, g2_hbm, out_hbm,
                  invb, rows, sbuf, obuf, g1b, g2b, sem):
    wid = lax.axis_index("s") * 2 + lax.axis_index("c")
    tok0 = wid * _TPW
    pltpu.sync_copy(g1_hbm.at[pl.ds(tok0, _TPW)], g1b)
    pltpu.sync_copy(g2_hbm.at[pl.ds(tok0, _TPW)], g2b)
    for c in range(_NCH):
        pltpu.sync_copy(inv_hbm.at[pl.ds(2 * tok0 + c * 2 * _CH, 2 * _CH)],
                        invb.at[c])
    for c in range(_NCH):
        pltpu.async_copy(os_hbm.at[invb.at[c]], rows, sem).wait()
        pltpu.sync_copy(os_hbm.at[pl.ds(T * K + tok0 + c * _CH, _CH)], sbuf)
        g1v = g1b[pl.ds(c * _CH, 16)]
        g2v = g2b[pl.ds(c * _CH, 16)]
        for j in range(_CH):
            ga = g1v[j]
            gb = g2v[j]

            def body(v, carry, j=j, ga=ga, gb=gb):
                sl = pl.ds(v * 16, 16)
                obuf[j, sl] = (ga * rows[2 * j, sl] + gb * rows[2 * j + 1, sl]
                               + sbuf[j, sl])
                return carry

            lax.fori_loop(0, D // 16, body, 0)
        pltpu.sync_copy(obuf, out_hbm.at[pl.ds(tok0 + c * _CH, _CH)])


def _combine(os, inv, g1, g2):
    f = functools.partial(
        pl.kernel,
        out_type=jax.ShapeDtypeStruct((T, D), jnp.float32),
        mesh=plsc.VectorSubcoreMesh(core_axis_name="c", subcore_axis_name="s"),
        scratch_types=[
            pltpu.VMEM((_NCH, 2 * _CH), jnp.int32),
            pltpu.VMEM((2 * _CH, D), jnp.float32),
            pltpu.VMEM((_CH, D), jnp.float32),
            pltpu.VMEM((_CH, D), jnp.float32),
            pltpu.VMEM((_TPW,), jnp.float32),
            pltpu.VMEM((_TPW,), jnp.float32),
            pltpu.SemaphoreType.DMA,
        ],
    )(_combine_body)
    return f(os, inv, g1, g2)


@jax.jit
def kernel(x, Wr, w1, w2, w3, sw1, sw2, sw3):
    logits = _router(x, Wr)

    m1 = jnp.max(logits, axis=-1)
    i1 = jnp.argmax(logits, axis=-1).astype(jnp.int32)
    masked = jnp.where(jax.nn.one_hot(i1, E, dtype=jnp.bool_), -jnp.inf, logits)
    i2 = jnp.argmax(masked, axis=-1).astype(jnp.int32)
    ex = jnp.exp(logits - m1[:, None])
    denom = jnp.sum(ex, axis=-1)
    probs = ex / denom[:, None]
    g1 = jnp.take_along_axis(probs, i1[:, None], axis=1)[:, 0]
    g2 = jnp.take_along_axis(probs, i2[:, None], axis=1)[:, 0]
    density = jnp.mean(jax.nn.one_hot(i1, E, dtype=jnp.float32), axis=0)
    prob_mean = jnp.mean(probs, axis=0)
    aux_loss = COEF * jnp.sum(density * prob_mean) * E

    flat_e = jnp.stack([i1, i2], axis=1).reshape(-1)

    perm = jnp.argsort(flat_e, stable=True).astype(jnp.int32)
    inv = jnp.argsort(perm, stable=True).astype(jnp.int32)
    counts = jnp.bincount(flat_e, length=E)
    offsets = jnp.concatenate([
        jnp.zeros((1,), jnp.int32),
        jnp.cumsum(counts).astype(jnp.int32),
        jnp.full((1,), R, jnp.int32),
    ])
    grp, blk, lo, hi, first = _tile_maps(offsets)

    ar = jnp.arange(T, dtype=jnp.int32)
    sorted_tok = jnp.concatenate([perm // K, ar])
    xs = x[sorted_tok]

    w1s = jnp.concatenate([w1, sw1[None]], axis=0)
    w3s = jnp.concatenate([w3, sw3[None]], axis=0)
    w2s = jnp.concatenate([w2, sw2[None]], axis=0)

    os = _grouped_ffn(xs, w1s, w3s, w2s, grp, blk, lo, hi, first)

    out = _combine(os, inv, g1.astype(jnp.float32), g2.astype(jnp.float32))
    return out, aux_loss


# speedup vs baseline: 4.8286x; 1.2190x over previous
import functools
import jax
import jax.numpy as jnp
from jax import lax
from jax.experimental import pallas as pl
from jax.experimental.pallas import tpu as pltpu
from jax.experimental.pallas import tpu_sc as plsc

T = 4096
D = 1024
H = 1024
E = 8
K = 2
COEF = 0.01

B = 256
R = T * K + T
NB = R // B
G = E + 1
NT = NB + G - 1


def _gelu(v):
    return 0.5 * v * (1.0 + jax.lax.erf(v * 0.7071067811865476))


def _router_body(x_ref, wr_ref, out_ref):
    out_ref[...] = jax.lax.dot_general(
        wr_ref[...], x_ref[...], (((1,), (1,)), ((), ())),
        preferred_element_type=jnp.float32)


def _router(x, Wr):
    blk = 512
    return pl.pallas_call(
        _router_body,
        grid=(T // blk,),
        in_specs=[
            pl.BlockSpec((blk, D), lambda i: (i, 0)),
            pl.BlockSpec((E, D), lambda i: (0, 0)),
        ],
        out_specs=pl.BlockSpec((E, blk), lambda i: (0, i)),
        out_shape=jax.ShapeDtypeStruct((E, T), jnp.float32),
    )(x, Wr)


def _ffn_body(grp_ref, blk_ref, lo_ref, hi_ref, first_ref,
              xs_ref, w1_ref, w3_ref, w2_ref, out_ref):
    i = pl.program_id(0)
    lo = lo_ref[i]
    hi = hi_ref[i]
    base = blk_ref[i] * B
    rows = base + jax.lax.broadcasted_iota(jnp.int32, (B, 1), 0)
    mask = (rows >= lo) & (rows < hi)

    x = xs_ref[...]
    h1 = jax.lax.dot_general(x, w1_ref[0], (((1,), (1,)), ((), ())),
                             preferred_element_type=jnp.float32)
    h3 = jax.lax.dot_general(x, w3_ref[0], (((1,), (1,)), ((), ())),
                             preferred_element_type=jnp.float32)
    h = _gelu(h1) * h3
    o = jax.lax.dot_general(h, w2_ref[0], (((1,), (1,)), ((), ())),
                            preferred_element_type=jnp.float32)
    o = jnp.where(mask, o, 0.0)

    @pl.when(first_ref[i] == 1)
    def _():
        out_ref[...] = o

    @pl.when(first_ref[i] == 0)
    def _():
        out_ref[...] += o


def _grouped_ffn(xs, w1s, w3s, w2s, grp, blk, lo, hi, first):
    grid_spec = pltpu.PrefetchScalarGridSpec(
        num_scalar_prefetch=5,
        grid=(NT,),
        in_specs=[
            pl.BlockSpec((B, D), lambda i, g, b, lo, hi, f: (b[i], 0)),
            pl.BlockSpec((1, H, D), lambda i, g, b, lo, hi, f: (g[i], 0, 0)),
            pl.BlockSpec((1, H, D), lambda i, g, b, lo, hi, f: (g[i], 0, 0)),
            pl.BlockSpec((1, D, H), lambda i, g, b, lo, hi, f: (g[i], 0, 0)),
        ],
        out_specs=pl.BlockSpec((B, D), lambda i, g, b, lo, hi, f: (b[i], 0)),
    )
    return pl.pallas_call(
        _ffn_body,
        grid_spec=grid_spec,
        out_shape=jax.ShapeDtypeStruct((R, D), jnp.float32),
    )(grp, blk, lo, hi, first, xs, w1s, w3s, w2s)


def _tile_maps(offsets):
    s = offsets[:-1]
    t = offsets[1:]
    nonempty = t > s
    bs = s // B
    be = jnp.where(nonempty, (t + B - 1) // B, bs)
    tiles_per = jnp.where(nonempty, be - bs, 0)
    start = jnp.concatenate([jnp.zeros((1,), jnp.int32),
                             jnp.cumsum(tiles_per)]).astype(jnp.int32)
    total = start[-1]
    j = jnp.arange(NT, dtype=jnp.int32)
    g = jnp.searchsorted(start[1:], j, side="right").astype(jnp.int32)
    g = jnp.minimum(g, G - 1)
    blk = bs[g] + (j - start[g])
    lo = jnp.maximum(s[g], blk * B)
    hi = jnp.minimum(t[g], (blk + 1) * B)
    valid = j < total
    last_blk = blk[jnp.maximum(total - 1, 0)]
    blk = jnp.where(valid, blk, last_blk)
    lo = jnp.where(valid, lo, 0)
    hi = jnp.where(valid, hi, 0)
    prev_blk = jnp.concatenate([jnp.full((1,), -1, jnp.int32), blk[:-1]])
    first = (valid & (blk != prev_blk)).astype(jnp.int32)
    return g, blk.astype(jnp.int32), lo.astype(jnp.int32), hi.astype(jnp.int32), first


_NW1 = 16
_TPW1 = T // _NW1
_RPW1 = _TPW1 * K
_NCH1 = _TPW1 // 16


def _dispatch_body(lg_hbm, inv0_hbm, inv1_hbm, g1_hbm, g2_hbm,
                   offs_hbm, aux_hbm,
                   lbuf, ebuf, invb, rankb, g1b, g2b, cntv,
                   psbuf, dnbuf, tabv, pstab, dntab, auxv,
                   cnt_sh, ps_sh, dn_sh, sem):
    cid = lax.axis_index("c")
    sid = lax.axis_index("s")
    active = cid == 0
    lane = lax.iota(jnp.int32, 16)

    @pl.when(active)
    def _phase1():
        tok0 = sid * _TPW1
        for e in range(E):
            pltpu.sync_copy(lg_hbm.at[pl.ds(e * T + tok0, _TPW1)],
                            lbuf.at[e])
        ps_acc = [jnp.zeros((16,), jnp.float32) for _ in range(E)]
        dn_acc = [jnp.zeros((16,), jnp.float32) for _ in range(E)]
        for j in range(_NCH1):
            sl = pl.ds(j * 16, 16)
            le = [lbuf[e, sl] for e in range(E)]
            m1 = le[0]
            i1 = jnp.zeros((16,), jnp.int32)
            for e in range(1, E):
                gt = le[e] > m1
                i1 = jnp.where(gt, jnp.int32(e), i1)
                m1 = jnp.where(gt, le[e], m1)
            m2 = jnp.full((16,), -3e38, jnp.float32)
            i2 = jnp.zeros((16,), jnp.int32)
            for e in range(E):
                gt = (le[e] > m2) & (i1 != e)
                i2 = jnp.where(gt, jnp.int32(e), i2)
                m2 = jnp.where(gt, le[e], m2)
            ex = [jnp.exp(le[e] - m1) for e in range(E)]
            sumexp = ex[0]
            for e in range(1, E):
                sumexp = sumexp + ex[e]
            g1 = 1.0 / sumexp
            g2 = jnp.exp(m2 - m1) * g1
            g1b[sl] = g1
            g2b[sl] = g2
            for e in range(E):
                ps_acc[e] = ps_acc[e] + ex[e] * g1
                dn_acc[e] = dn_acc[e] + jnp.where(i1 == e, 1.0, 0.0)
            ebuf[sl] = i1
            ebuf[pl.ds(_TPW1 + j * 16, 16)] = i2
        for e in range(E):
            psbuf[e] = ps_acc[e]
            dnbuf[e] = dn_acc[e]
        carries = [jnp.int32(0) for _ in range(E)]
        for v in range(_RPW1 // 16):
            ev = ebuf[pl.ds(v * 16, 16)]
            rank = jnp.zeros((16,), jnp.int32)
            for e in range(E):
                m = ev == e
                mi = jnp.where(m, jnp.int32(1), jnp.int32(0))
                cs = plsc.cumsum(mi)
                rank = jnp.where(m, carries[e] + cs - 1, rank)
                carries[e] = carries[e] + jnp.sum(mi)
            rankb[pl.ds(v * 16, 16)] = rank
        cvec = jnp.zeros((16,), jnp.int32)
        for e in range(E):
            cvec = jnp.where(lane == e, carries[e], cvec)
        cntv[...] = cvec
        pltpu.sync_copy(cntv, cnt_sh.at[sid])
        pltpu.sync_copy(psbuf, ps_sh.at[sid])
        pltpu.sync_copy(dnbuf, dn_sh.at[sid])

    plsc.subcore_barrier()

    @pl.when(active)
    def _phase2():
        tok0 = sid * _TPW1
        pltpu.sync_copy(cnt_sh, tabv)
        totals = jnp.zeros((16,), jnp.int32)
        myprefix = jnp.zeros((16,), jnp.int32)
        sidv = jnp.broadcast_to(sid, (16,))
        for w in range(_NW1):
            row = tabv[w]
            totals = totals + row
            before = jnp.full((16,), w, jnp.int32) < sidv
            myprefix = myprefix + jnp.where(before, row, 0)
        excl = plsc.cumsum(totals) - totals
        base = excl + myprefix
        for v in range(_RPW1 // 16):
            sl = pl.ds(v * 16, 16)
            ev = ebuf[sl]
            dst = rankb[sl]
            for e in range(E):
                dst = jnp.where(ev == e, dst + base[e], dst)
            invb[sl] = dst
        pltpu.sync_copy(invb.at[pl.ds(0, _TPW1)],
                        inv0_hbm.at[pl.ds(tok0, _TPW1)])
        pltpu.sync_copy(invb.at[pl.ds(_TPW1, _TPW1)],
                        inv1_hbm.at[pl.ds(tok0, _TPW1)])
        pltpu.sync_copy(g1b, g1_hbm.at[pl.ds(tok0, _TPW1)])
        pltpu.sync_copy(g2b, g2_hbm.at[pl.ds(tok0, _TPW1)])

        @pl.when(sid == 0)
        def _finalize():
            cntv[...] = excl
            pltpu.sync_copy(cntv, offs_hbm)
            pltpu.sync_copy(ps_sh, pstab)
            pltpu.sync_copy(dn_sh, dntab)
            aux = jnp.float32(0.0)
            for e in range(E):
                pse = jnp.zeros((16,), jnp.float32)
                dne = jnp.zeros((16,), jnp.float32)
                for w in range(_NW1):
                    pse = pse + pstab[w, e]
                    dne = dne + dntab[w, e]
                aux = aux + jnp.sum(pse) * jnp.sum(dne)
            aux = aux * (COEF * E / (T * T))
            auxv[...] = jnp.where(lane == 0, aux, 0.0)
            pltpu.sync_copy(auxv, aux_hbm)


def _dispatch(logits_flat):
    f = functools.partial(
        pl.kernel,
        out_type=(
            jax.ShapeDtypeStruct((T,), jnp.int32),
            jax.ShapeDtypeStruct((T,), jnp.int32),
            jax.ShapeDtypeStruct((T,), jnp.float32),
            jax.ShapeDtypeStruct((T,), jnp.float32),
            jax.ShapeDtypeStruct((16,), jnp.int32),
            jax.ShapeDtypeStruct((16,), jnp.float32),
        ),
        mesh=plsc.VectorSubcoreMesh(core_axis_name="c", subcore_axis_name="s"),
        compiler_params=pltpu.CompilerParams(needs_layout_passes=False,
                                             use_tc_tiling_on_sc=False),
        scratch_types=[
            pltpu.VMEM((E, _TPW1), jnp.float32),
            pltpu.VMEM((_RPW1,), jnp.int32),
            pltpu.VMEM((_RPW1,), jnp.int32),
            pltpu.VMEM((_RPW1,), jnp.int32),
            pltpu.VMEM((_TPW1,), jnp.float32),
            pltpu.VMEM((_TPW1,), jnp.float32),
            pltpu.VMEM((16,), jnp.int32),
            pltpu.VMEM((E, 16), jnp.float32),
            pltpu.VMEM((E, 16), jnp.float32),
            pltpu.VMEM((_NW1, 16), jnp.int32),
            pltpu.VMEM((_NW1, E, 16), jnp.float32),
            pltpu.VMEM((_NW1, E, 16), jnp.float32),
            pltpu.VMEM((16,), jnp.float32),
            pltpu.VMEM_SHARED((_NW1, 16), jnp.int32),
            pltpu.VMEM_SHARED((_NW1, E, 16), jnp.float32),
            pltpu.VMEM_SHARED((_NW1, E, 16), jnp.float32),
            pltpu.SemaphoreType.DMA,
        ],
    )(_dispatch_body)
    return f(logits_flat)


def _xscatter_body(x_hbm, inv0_hbm, inv1_hbm, xs_hbm, xbuf, d0, d1, sem):
    wid = lax.axis_index("s") * 2 + lax.axis_index("c")
    tok0 = wid * _TPW
    for c in range(4):
        pltpu.sync_copy(inv0_hbm.at[pl.ds(tok0 + 32 * c, 32)], d0.at[c])
        pltpu.sync_copy(inv1_hbm.at[pl.ds(tok0 + 32 * c, 32)], d1.at[c])
    for c in range(4):
        pltpu.sync_copy(x_hbm.at[pl.ds(tok0 + 32 * c, 32)], xbuf)
        a = pltpu.async_copy(xbuf, xs_hbm.at[d0.at[c]], sem)
        b = pltpu.async_copy(xbuf, xs_hbm.at[d1.at[c]], sem)
        pltpu.sync_copy(xbuf, xs_hbm.at[pl.ds(T * K + tok0 + 32 * c, 32)])
        a.wait()
        b.wait()


def _xscatter(x, inv0, inv1):
    f = functools.partial(
        pl.kernel,
        out_type=jax.ShapeDtypeStruct((R, D), jnp.float32),
        mesh=plsc.VectorSubcoreMesh(core_axis_name="c", subcore_axis_name="s"),
        compiler_params=pltpu.CompilerParams(needs_layout_passes=False),
        scratch_types=[
            pltpu.VMEM((32, D), jnp.float32),
            pltpu.VMEM((4, 32), jnp.int32),
            pltpu.VMEM((4, 32), jnp.int32),
            pltpu.SemaphoreType.DMA,
        ],
    )(_xscatter_body)
    return f(x, inv0, inv1)


_NW = 32
_TPW = T // _NW
_CH = 16
_NCH = _TPW // _CH


def _combine_body(os_hbm, inv0_hbm, inv1_hbm, g1_hbm, g2_hbm, out_hbm,
                  i0b, i1b, rows0, rows1, sbuf, obuf, g1b, g2b, sem):
    wid = lax.axis_index("s") * 2 + lax.axis_index("c")
    tok0 = wid * _TPW
    pltpu.sync_copy(g1_hbm.at[pl.ds(tok0, _TPW)], g1b)
    pltpu.sync_copy(g2_hbm.at[pl.ds(tok0, _TPW)], g2b)
    for c in range(_NCH):
        pltpu.sync_copy(inv0_hbm.at[pl.ds(tok0 + c * _CH, _CH)], i0b.at[c])
        pltpu.sync_copy(inv1_hbm.at[pl.ds(tok0 + c * _CH, _CH)], i1b.at[c])
    for c in range(_NCH):
        a = pltpu.async_copy(os_hbm.at[i0b.at[c]], rows0, sem)
        b = pltpu.async_copy(os_hbm.at[i1b.at[c]], rows1, sem)
        pltpu.sync_copy(os_hbm.at[pl.ds(T * K + tok0 + c * _CH, _CH)], sbuf)
        a.wait()
        b.wait()
        g1v = g1b[pl.ds(c * _CH, 16)]
        g2v = g2b[pl.ds(c * _CH, 16)]
        for j in range(_CH):
            ga = g1v[j]
            gb = g2v[j]

            def body(v, carry, j=j, ga=ga, gb=gb):
                sl = pl.ds(v * 16, 16)
                obuf[j, sl] = (ga * rows0[j, sl] + gb * rows1[j, sl]
                               + sbuf[j, sl])
                return carry

            lax.fori_loop(0, D // 16, body, 0)
        pltpu.sync_copy(obuf, out_hbm.at[pl.ds(tok0 + c * _CH, _CH)])


def _combine(os, inv0, inv1, g1, g2):
    f = functools.partial(
        pl.kernel,
        out_type=jax.ShapeDtypeStruct((T, D), jnp.float32),
        mesh=plsc.VectorSubcoreMesh(core_axis_name="c", subcore_axis_name="s"),
        compiler_params=pltpu.CompilerParams(needs_layout_passes=False),
        scratch_types=[
            pltpu.VMEM((_NCH, _CH), jnp.int32),
            pltpu.VMEM((_NCH, _CH), jnp.int32),
            pltpu.VMEM((_CH, D), jnp.float32),
            pltpu.VMEM((_CH, D), jnp.float32),
            pltpu.VMEM((_CH, D), jnp.float32),
            pltpu.VMEM((_CH, D), jnp.float32),
            pltpu.VMEM((_TPW,), jnp.float32),
            pltpu.VMEM((_TPW,), jnp.float32),
            pltpu.SemaphoreType.DMA,
        ],
    )(_combine_body)
    return f(os, inv0, inv1, g1, g2)


@jax.jit
def kernel(x, Wr, w1, w2, w3, sw1, sw2, sw3):
    logits = _router(x, Wr)

    inv0, inv1, g1, g2, offs, aux16 = _dispatch(logits.reshape(-1))
    aux_loss = aux16[0]

    offsets = jnp.concatenate([offs[:G], jnp.full((1,), R, jnp.int32)])
    grp, blk, lo, hi, first = _tile_maps(offsets)

    xs = _xscatter(x, inv0, inv1)

    w1s = jnp.concatenate([w1, sw1[None]], axis=0)
    w3s = jnp.concatenate([w3, sw3[None]], axis=0)
    w2s = jnp.concatenate([w2, sw2[None]], axis=0)

    os = _grouped_ffn(xs, w1s, w3s, w2s, grp, blk, lo, hi, first)

    out = _combine(os, inv0, inv1, g1, g2)
    return out, aux_loss


# speedup vs baseline: 4.9103x; 1.0169x over previous
import functools
import jax
import jax.numpy as jnp
from jax import lax
from jax.experimental import pallas as pl
from jax.experimental.pallas import tpu as pltpu
from jax.experimental.pallas import tpu_sc as plsc

T = 4096
D = 1024
H = 1024
E = 8
K = 2
COEF = 0.01

B = 256
R = T * K + T
NB = R // B
G = E + 1
NT = NB + G - 1


def _gelu(v):
    return 0.5 * v * (1.0 + jax.lax.erf(v * 0.7071067811865476))


def _router_body(x_ref, wr_ref, out_ref):
    out_ref[...] = jax.lax.dot_general(
        wr_ref[...], x_ref[...], (((1,), (1,)), ((), ())),
        preferred_element_type=jnp.float32)


def _router(x, Wr):
    blk = 512
    return pl.pallas_call(
        _router_body,
        grid=(T // blk,),
        in_specs=[
            pl.BlockSpec((blk, D), lambda i: (i, 0)),
            pl.BlockSpec((E, D), lambda i: (0, 0)),
        ],
        out_specs=pl.BlockSpec((E, blk), lambda i: (0, i)),
        out_shape=jax.ShapeDtypeStruct((E, T), jnp.float32),
    )(x, Wr)


def _ffn_body(grp_ref, blk_ref, lo_ref, hi_ref, first_ref,
              xs_ref, w1_ref, w3_ref, w2_ref, out_ref):
    i = pl.program_id(0)
    lo = lo_ref[i]
    hi = hi_ref[i]
    base = blk_ref[i] * B
    rows = base + jax.lax.broadcasted_iota(jnp.int32, (B, 1), 0)
    mask = (rows >= lo) & (rows < hi)

    x = xs_ref[...].astype(jnp.bfloat16)
    h1 = jax.lax.dot_general(x, w1_ref[0], (((1,), (1,)), ((), ())),
                             preferred_element_type=jnp.float32)
    h3 = jax.lax.dot_general(x, w3_ref[0], (((1,), (1,)), ((), ())),
                             preferred_element_type=jnp.float32)
    h = (_gelu(h1) * h3).astype(jnp.bfloat16)
    o = jax.lax.dot_general(h, w2_ref[0], (((1,), (1,)), ((), ())),
                            preferred_element_type=jnp.float32)
    o = jnp.where(mask, o, 0.0)

    @pl.when(first_ref[i] == 1)
    def _():
        out_ref[...] = o

    @pl.when(first_ref[i] == 0)
    def _():
        out_ref[...] += o


def _grouped_ffn(xs, w1s, w3s, w2s, grp, blk, lo, hi, first):
    grid_spec = pltpu.PrefetchScalarGridSpec(
        num_scalar_prefetch=5,
        grid=(NT,),
        in_specs=[
            pl.BlockSpec((B, D), lambda i, g, b, lo, hi, f: (b[i], 0)),
            pl.BlockSpec((1, H, D), lambda i, g, b, lo, hi, f: (g[i], 0, 0)),
            pl.BlockSpec((1, H, D), lambda i, g, b, lo, hi, f: (g[i], 0, 0)),
            pl.BlockSpec((1, D, H), lambda i, g, b, lo, hi, f: (g[i], 0, 0)),
        ],
        out_specs=pl.BlockSpec((B, D), lambda i, g, b, lo, hi, f: (b[i], 0)),
    )
    return pl.pallas_call(
        _ffn_body,
        grid_spec=grid_spec,
        out_shape=jax.ShapeDtypeStruct((R, D), jnp.float32),
    )(grp, blk, lo, hi, first, xs, w1s, w3s, w2s)


def _tile_maps(offsets):
    s = offsets[:-1]
    t = offsets[1:]
    nonempty = t > s
    bs = s // B
    be = jnp.where(nonempty, (t + B - 1) // B, bs)
    tiles_per = jnp.where(nonempty, be - bs, 0)
    start = jnp.concatenate([jnp.zeros((1,), jnp.int32),
                             jnp.cumsum(tiles_per)]).astype(jnp.int32)
    total = start[-1]
    j = jnp.arange(NT, dtype=jnp.int32)
    g = jnp.searchsorted(start[1:], j, side="right").astype(jnp.int32)
    g = jnp.minimum(g, G - 1)
    blk = bs[g] + (j - start[g])
    lo = jnp.maximum(s[g], blk * B)
    hi = jnp.minimum(t[g], (blk + 1) * B)
    valid = j < total
    last_blk = blk[jnp.maximum(total - 1, 0)]
    blk = jnp.where(valid, blk, last_blk)
    lo = jnp.where(valid, lo, 0)
    hi = jnp.where(valid, hi, 0)
    prev_blk = jnp.concatenate([jnp.full((1,), -1, jnp.int32), blk[:-1]])
    first = (valid & (blk != prev_blk)).astype(jnp.int32)
    return g, blk.astype(jnp.int32), lo.astype(jnp.int32), hi.astype(jnp.int32), first


_NW1 = 16
_TPW1 = T // _NW1
_RPW1 = _TPW1 * K
_NCH1 = _TPW1 // 16


def _dispatch_body(lg_hbm, inv0_hbm, inv1_hbm, g1_hbm, g2_hbm,
                   offs_hbm, aux_hbm,
                   lbuf, ebuf, invb, rankb, g1b, g2b, cntv,
                   psbuf, dnbuf, tabv, pstab, dntab, auxv,
                   cnt_sh, ps_sh, dn_sh, sem):
    cid = lax.axis_index("c")
    sid = lax.axis_index("s")
    active = cid == 0
    lane = lax.iota(jnp.int32, 16)

    @pl.when(active)
    def _phase1():
        tok0 = sid * _TPW1
        for e in range(E):
            pltpu.sync_copy(lg_hbm.at[pl.ds(e * T + tok0, _TPW1)],
                            lbuf.at[e])
        ps_acc = [jnp.zeros((16,), jnp.float32) for _ in range(E)]
        dn_acc = [jnp.zeros((16,), jnp.float32) for _ in range(E)]
        for j in range(_NCH1):
            sl = pl.ds(j * 16, 16)
            le = [lbuf[e, sl] for e in range(E)]
            m1 = le[0]
            i1 = jnp.zeros((16,), jnp.int32)
            for e in range(1, E):
                gt = le[e] > m1
                i1 = jnp.where(gt, jnp.int32(e), i1)
                m1 = jnp.where(gt, le[e], m1)
            m2 = jnp.full((16,), -3e38, jnp.float32)
            i2 = jnp.zeros((16,), jnp.int32)
            for e in range(E):
                gt = (le[e] > m2) & (i1 != e)
                i2 = jnp.where(gt, jnp.int32(e), i2)
                m2 = jnp.where(gt, le[e], m2)
            ex = [jnp.exp(le[e] - m1) for e in range(E)]
            sumexp = ex[0]
            for e in range(1, E):
                sumexp = sumexp + ex[e]
            g1 = 1.0 / sumexp
            g2 = jnp.exp(m2 - m1) * g1
            g1b[sl] = g1
            g2b[sl] = g2
            for e in range(E):
                ps_acc[e] = ps_acc[e] + ex[e] * g1
                dn_acc[e] = dn_acc[e] + jnp.where(i1 == e, 1.0, 0.0)
            ebuf[sl] = i1
            ebuf[pl.ds(_TPW1 + j * 16, 16)] = i2
        for e in range(E):
            psbuf[e] = ps_acc[e]
            dnbuf[e] = dn_acc[e]
        carries = [jnp.int32(0) for _ in range(E)]
        for v in range(_RPW1 // 16):
            ev = ebuf[pl.ds(v * 16, 16)]
            rank = jnp.zeros((16,), jnp.int32)
            for e in range(E):
                m = ev == e
                mi = jnp.where(m, jnp.int32(1), jnp.int32(0))
                cs = plsc.cumsum(mi)
                rank = jnp.where(m, carries[e] + cs - 1, rank)
                carries[e] = carries[e] + jnp.sum(mi)
            rankb[pl.ds(v * 16, 16)] = rank
        cvec = jnp.zeros((16,), jnp.int32)
        for e in range(E):
            cvec = jnp.where(lane == e, carries[e], cvec)
        cntv[...] = cvec
        pltpu.sync_copy(cntv, cnt_sh.at[sid])
        pltpu.sync_copy(psbuf, ps_sh.at[sid])
        pltpu.sync_copy(dnbuf, dn_sh.at[sid])

    plsc.subcore_barrier()

    @pl.when(active)
    def _phase2():
        tok0 = sid * _TPW1
        pltpu.sync_copy(cnt_sh, tabv)
        totals = jnp.zeros((16,), jnp.int32)
        myprefix = jnp.zeros((16,), jnp.int32)
        sidv = jnp.broadcast_to(sid, (16,))
        for w in range(_NW1):
            row = tabv[w]
            totals = totals + row
            before = jnp.full((16,), w, jnp.int32) < sidv
            myprefix = myprefix + jnp.where(before, row, 0)
        excl = plsc.cumsum(totals) - totals
        base = excl + myprefix
        for v in range(_RPW1 // 16):
            sl = pl.ds(v * 16, 16)
            ev = ebuf[sl]
            dst = rankb[sl]
            for e in range(E):
                dst = jnp.where(ev == e, dst + base[e], dst)
            invb[sl] = dst
        pltpu.sync_copy(invb.at[pl.ds(0, _TPW1)],
                        inv0_hbm.at[pl.ds(tok0, _TPW1)])
        pltpu.sync_copy(invb.at[pl.ds(_TPW1, _TPW1)],
                        inv1_hbm.at[pl.ds(tok0, _TPW1)])
        pltpu.sync_copy(g1b, g1_hbm.at[pl.ds(tok0, _TPW1)])
        pltpu.sync_copy(g2b, g2_hbm.at[pl.ds(tok0, _TPW1)])

        @pl.when(sid == 0)
        def _finalize():
            cntv[...] = excl
            pltpu.sync_copy(cntv, offs_hbm)
            pltpu.sync_copy(ps_sh, pstab)
            pltpu.sync_copy(dn_sh, dntab)
            aux = jnp.float32(0.0)
            for e in range(E):
                pse = jnp.zeros((16,), jnp.float32)
                dne = jnp.zeros((16,), jnp.float32)
                for w in range(_NW1):
                    pse = pse + pstab[w, e]
                    dne = dne + dntab[w, e]
                aux = aux + jnp.sum(pse) * jnp.sum(dne)
            aux = aux * (COEF * E / (T * T))
            auxv[...] = jnp.where(lane == 0, aux, 0.0)
            pltpu.sync_copy(auxv, aux_hbm)


def _dispatch(logits_flat):
    f = functools.partial(
        pl.kernel,
        out_type=(
            jax.ShapeDtypeStruct((T,), jnp.int32),
            jax.ShapeDtypeStruct((T,), jnp.int32),
            jax.ShapeDtypeStruct((T,), jnp.float32),
            jax.ShapeDtypeStruct((T,), jnp.float32),
            jax.ShapeDtypeStruct((16,), jnp.int32),
            jax.ShapeDtypeStruct((16,), jnp.float32),
        ),
        mesh=plsc.VectorSubcoreMesh(core_axis_name="c", subcore_axis_name="s"),
        compiler_params=pltpu.CompilerParams(needs_layout_passes=False,
                                             use_tc_tiling_on_sc=False),
        scratch_types=[
            pltpu.VMEM((E, _TPW1), jnp.float32),
            pltpu.VMEM((_RPW1,), jnp.int32),
            pltpu.VMEM((_RPW1,), jnp.int32),
            pltpu.VMEM((_RPW1,), jnp.int32),
            pltpu.VMEM((_TPW1,), jnp.float32),
            pltpu.VMEM((_TPW1,), jnp.float32),
            pltpu.VMEM((16,), jnp.int32),
            pltpu.VMEM((E, 16), jnp.float32),
            pltpu.VMEM((E, 16), jnp.float32),
            pltpu.VMEM((_NW1, 16), jnp.int32),
            pltpu.VMEM((_NW1, E, 16), jnp.float32),
            pltpu.VMEM((_NW1, E, 16), jnp.float32),
            pltpu.VMEM((16,), jnp.float32),
            pltpu.VMEM_SHARED((_NW1, 16), jnp.int32),
            pltpu.VMEM_SHARED((_NW1, E, 16), jnp.float32),
            pltpu.VMEM_SHARED((_NW1, E, 16), jnp.float32),
            pltpu.SemaphoreType.DMA,
        ],
    )(_dispatch_body)
    return f(logits_flat)


def _xscatter_body(x_hbm, inv0_hbm, inv1_hbm, xs_hbm, xbuf, d0, d1, sem):
    wid = lax.axis_index("s") * 2 + lax.axis_index("c")
    tok0 = wid * _TPW
    for c in range(4):
        pltpu.sync_copy(inv0_hbm.at[pl.ds(tok0 + 32 * c, 32)], d0.at[c])
        pltpu.sync_copy(inv1_hbm.at[pl.ds(tok0 + 32 * c, 32)], d1.at[c])
    for c in range(4):
        pltpu.sync_copy(x_hbm.at[pl.ds(tok0 + 32 * c, 32)], xbuf)
        a = pltpu.async_copy(xbuf, xs_hbm.at[d0.at[c]], sem)
        b = pltpu.async_copy(xbuf, xs_hbm.at[d1.at[c]], sem)
        pltpu.sync_copy(xbuf, xs_hbm.at[pl.ds(T * K + tok0 + 32 * c, 32)])
        a.wait()
        b.wait()


def _xscatter(x, inv0, inv1):
    f = functools.partial(
        pl.kernel,
        out_type=jax.ShapeDtypeStruct((R, D), jnp.float32),
        mesh=plsc.VectorSubcoreMesh(core_axis_name="c", subcore_axis_name="s"),
        compiler_params=pltpu.CompilerParams(needs_layout_passes=False),
        scratch_types=[
            pltpu.VMEM((32, D), jnp.float32),
            pltpu.VMEM((4, 32), jnp.int32),
            pltpu.VMEM((4, 32), jnp.int32),
            pltpu.SemaphoreType.DMA,
        ],
    )(_xscatter_body)
    return f(x, inv0, inv1)


_NW = 32
_TPW = T // _NW
_CH = 16
_NCH = _TPW // _CH


def _combine_body(os_hbm, inv0_hbm, inv1_hbm, g1_hbm, g2_hbm, out_hbm,
                  i0b, i1b, rows0, rows1, sbuf, obuf, g1b, g2b, sem):
    wid = lax.axis_index("s") * 2 + lax.axis_index("c")
    tok0 = wid * _TPW
    pltpu.sync_copy(g1_hbm.at[pl.ds(tok0, _TPW)], g1b)
    pltpu.sync_copy(g2_hbm.at[pl.ds(tok0, _TPW)], g2b)
    for c in range(_NCH):
        pltpu.sync_copy(inv0_hbm.at[pl.ds(tok0 + c * _CH, _CH)], i0b.at[c])
        pltpu.sync_copy(inv1_hbm.at[pl.ds(tok0 + c * _CH, _CH)], i1b.at[c])
    for c in range(_NCH):
        a = pltpu.async_copy(os_hbm.at[i0b.at[c]], rows0, sem)
        b = pltpu.async_copy(os_hbm.at[i1b.at[c]], rows1, sem)
        pltpu.sync_copy(os_hbm.at[pl.ds(T * K + tok0 + c * _CH, _CH)], sbuf)
        a.wait()
        b.wait()
        g1v = g1b[pl.ds(c * _CH, 16)]
        g2v = g2b[pl.ds(c * _CH, 16)]
        for j in range(_CH):
            ga = g1v[j]
            gb = g2v[j]

            def body(v, carry, j=j, ga=ga, gb=gb):
                sl = pl.ds(v * 16, 16)
                obuf[j, sl] = (ga * rows0[j, sl] + gb * rows1[j, sl]
                               + sbuf[j, sl])
                return carry

            lax.fori_loop(0, D // 16, body, 0)
        pltpu.sync_copy(obuf, out_hbm.at[pl.ds(tok0 + c * _CH, _CH)])


def _combine(os, inv0, inv1, g1, g2):
    f = functools.partial(
        pl.kernel,
        out_type=jax.ShapeDtypeStruct((T, D), jnp.float32),
        mesh=plsc.VectorSubcoreMesh(core_axis_name="c", subcore_axis_name="s"),
        compiler_params=pltpu.CompilerParams(needs_layout_passes=False),
        scratch_types=[
            pltpu.VMEM((_NCH, _CH), jnp.int32),
            pltpu.VMEM((_NCH, _CH), jnp.int32),
            pltpu.VMEM((_CH, D), jnp.float32),
            pltpu.VMEM((_CH, D), jnp.float32),
            pltpu.VMEM((_CH, D), jnp.float32),
            pltpu.VMEM((_CH, D), jnp.float32),
            pltpu.VMEM((_TPW,), jnp.float32),
            pltpu.VMEM((_TPW,), jnp.float32),
            pltpu.SemaphoreType.DMA,
        ],
    )(_combine_body)
    return f(os, inv0, inv1, g1, g2)


@jax.jit
def kernel(x, Wr, w1, w2, w3, sw1, sw2, sw3):
    logits = _router(x, Wr)

    inv0, inv1, g1, g2, offs, aux16 = _dispatch(logits.reshape(-1))
    aux_loss = aux16[0]

    offsets = jnp.concatenate([offs[:G], jnp.full((1,), R, jnp.int32)])
    grp, blk, lo, hi, first = _tile_maps(offsets)

    xs = _xscatter(x, inv0, inv1)

    w1s = jnp.concatenate([w1, sw1[None]], axis=0).astype(jnp.bfloat16)
    w3s = jnp.concatenate([w3, sw3[None]], axis=0).astype(jnp.bfloat16)
    w2s = jnp.concatenate([w2, sw2[None]], axis=0).astype(jnp.bfloat16)

    os = _grouped_ffn(xs, w1s, w3s, w2s, grp, blk, lo, hi, first)

    out = _combine(os, inv0, inv1, g1, g2)
    return out, aux_loss


# speedup vs baseline: 6.9722x; 1.4199x over previous
import functools
import jax
import jax.numpy as jnp
from jax import lax
from jax.experimental import pallas as pl
from jax.experimental.pallas import tpu as pltpu
from jax.experimental.pallas import tpu_sc as plsc

T = 4096
D = 1024
H = 1024
E = 8
K = 2
COEF = 0.01

B = 256
R = T * K
NB = R // B
NT = NB + E - 1
TM = 48


def _gelu(v):
    return 0.5 * v * (1.0 + jax.lax.erf(v * 0.7071067811865476))


def _router_body(x_ref, wr_ref, out_ref):
    out_ref[...] = jax.lax.dot_general(
        wr_ref[...], x_ref[...], (((1,), (1,)), ((), ())),
        preferred_element_type=jnp.float32)


def _router(x, Wr):
    blk = 512
    return pl.pallas_call(
        _router_body,
        grid=(T // blk,),
        in_specs=[
            pl.BlockSpec((blk, D), lambda i: (i, 0)),
            pl.BlockSpec((E, D), lambda i: (0, 0)),
        ],
        out_specs=pl.BlockSpec((E, blk), lambda i: (0, i)),
        out_shape=jax.ShapeDtypeStruct((E, T), jnp.float32),
    )(x, Wr)


def _ffn_body(grp_ref, blk_ref, lo_ref, hi_ref, first_ref, wchg_ref,
              xs_ref, w1_ref, w3_ref, w2_ref, out_ref, ws1, ws3, ws2):
    i = pl.program_id(0)
    lo = lo_ref[i]
    hi = hi_ref[i]
    base = blk_ref[i] * B
    rows = base + jax.lax.broadcasted_iota(jnp.int32, (B, 1), 0)
    mask = (rows >= lo) & (rows < hi)

    @pl.when(wchg_ref[i] == 1)
    def _():
        ws1[...] = w1_ref[0].astype(jnp.bfloat16)
        ws3[...] = w3_ref[0].astype(jnp.bfloat16)
        ws2[...] = w2_ref[0].astype(jnp.bfloat16)

    x = xs_ref[...].astype(jnp.bfloat16)
    h1 = jax.lax.dot_general(x, ws1[...], (((1,), (1,)), ((), ())),
                             preferred_element_type=jnp.float32)
    h3 = jax.lax.dot_general(x, ws3[...], (((1,), (1,)), ((), ())),
                             preferred_element_type=jnp.float32)
    h = (_gelu(h1) * h3).astype(jnp.bfloat16)
    o = jax.lax.dot_general(h, ws2[...], (((1,), (1,)), ((), ())),
                            preferred_element_type=jnp.float32)
    o = jnp.where(mask, o, 0.0)

    @pl.when(first_ref[i] == 1)
    def _():
        out_ref[...] = o

    @pl.when(first_ref[i] == 0)
    def _():
        out_ref[...] += o


def _grouped_ffn(xs, w1, w3, w2, grp, blk, lo, hi, first, wchg):
    grid_spec = pltpu.PrefetchScalarGridSpec(
        num_scalar_prefetch=6,
        grid=(NT,),
        in_specs=[
            pl.BlockSpec((B, D), lambda i, g, b, lo, hi, f, w: (b[i], 0)),
            pl.BlockSpec((1, H, D), lambda i, g, b, lo, hi, f, w: (g[i], 0, 0)),
            pl.BlockSpec((1, H, D), lambda i, g, b, lo, hi, f, w: (g[i], 0, 0)),
            pl.BlockSpec((1, D, H), lambda i, g, b, lo, hi, f, w: (g[i], 0, 0)),
        ],
        out_specs=pl.BlockSpec((B, D), lambda i, g, b, lo, hi, f, w: (b[i], 0)),
        scratch_shapes=[
            pltpu.VMEM((H, D), jnp.bfloat16),
            pltpu.VMEM((H, D), jnp.bfloat16),
            pltpu.VMEM((D, H), jnp.bfloat16),
        ],
    )
    return pl.pallas_call(
        _ffn_body,
        grid_spec=grid_spec,
        out_shape=jax.ShapeDtypeStruct((R, D), jnp.float32),
    )(grp, blk, lo, hi, first, wchg, xs, w1, w3, w2)


def _sffn_body(x_ref, w1_ref, w3_ref, w2_ref, out_ref):
    x = x_ref[...].astype(jnp.bfloat16)
    h1 = jax.lax.dot_general(x, w1_ref[...], (((1,), (1,)), ((), ())),
                             preferred_element_type=jnp.float32)
    h3 = jax.lax.dot_general(x, w3_ref[...], (((1,), (1,)), ((), ())),
                             preferred_element_type=jnp.float32)
    h = (_gelu(h1) * h3).astype(jnp.bfloat16)
    out_ref[...] = jax.lax.dot_general(h, w2_ref[...], (((1,), (1,)), ((), ())),
                                       preferred_element_type=jnp.float32)


def _shared_ffn(x, sw1b, sw3b, sw2b):
    blk = 512
    return pl.pallas_call(
        _sffn_body,
        grid=(T // blk,),
        in_specs=[
            pl.BlockSpec((blk, D), lambda i: (i, 0)),
            pl.BlockSpec((H, D), lambda i: (0, 0)),
            pl.BlockSpec((H, D), lambda i: (0, 0)),
            pl.BlockSpec((D, H), lambda i: (0, 0)),
        ],
        out_specs=pl.BlockSpec((blk, D), lambda i: (i, 0)),
        out_shape=jax.ShapeDtypeStruct((T, D), jnp.float32),
    )(x, sw1b, sw3b, sw2b)


_NW1 = 16
_TPW1 = T // _NW1
_RPW1 = _TPW1 * K
_NCH1 = _TPW1 // 16


def _dispatch_body(lg_hbm, inv0_hbm, inv1_hbm, g1_hbm, g2_hbm,
                   aux_hbm, grp_hbm, blk_hbm, lo_hbm, hi_hbm,
                   first_hbm, wchg_hbm,
                   lbuf, ebuf, invb, rankb, g1b, g2b, cntv,
                   psbuf, dnbuf, tabv, pstab, dntab, auxv, tmbuf,
                   cnt_sh, ps_sh, dn_sh, sem):
    cid = lax.axis_index("c")
    sid = lax.axis_index("s")
    active = cid == 0
    lane = lax.iota(jnp.int32, 16)

    @pl.when(active)
    def _phase1():
        tok0 = sid * _TPW1
        for e in range(E):
            pltpu.sync_copy(lg_hbm.at[pl.ds(e * T + tok0, _TPW1)],
                            lbuf.at[e])
        ps_acc = [jnp.zeros((16,), jnp.float32) for _ in range(E)]
        dn_acc = [jnp.zeros((16,), jnp.float32) for _ in range(E)]
        for j in range(_NCH1):
            sl = pl.ds(j * 16, 16)
            le = [lbuf[e, sl] for e in range(E)]
            m1 = le[0]
            i1 = jnp.zeros((16,), jnp.int32)
            for e in range(1, E):
                gt = le[e] > m1
                i1 = jnp.where(gt, jnp.int32(e), i1)
                m1 = jnp.where(gt, le[e], m1)
            m2 = jnp.full((16,), -3e38, jnp.float32)
            i2 = jnp.zeros((16,), jnp.int32)
            for e in range(E):
                gt = (le[e] > m2) & (i1 != e)
                i2 = jnp.where(gt, jnp.int32(e), i2)
                m2 = jnp.where(gt, le[e], m2)
            ex = [jnp.exp(le[e] - m1) for e in range(E)]
            sumexp = ex[0]
            for e in range(1, E):
                sumexp = sumexp + ex[e]
            g1 = 1.0 / sumexp
            g2 = jnp.exp(m2 - m1) * g1
            g1b[sl] = g1
            g2b[sl] = g2
            for e in range(E):
                ps_acc[e] = ps_acc[e] + ex[e] * g1
                dn_acc[e] = dn_acc[e] + jnp.where(i1 == e, 1.0, 0.0)
            ebuf[sl] = i1
            ebuf[pl.ds(_TPW1 + j * 16, 16)] = i2
        for e in range(E):
            psbuf[e] = ps_acc[e]
            dnbuf[e] = dn_acc[e]
        carries = [jnp.int32(0) for _ in range(E)]
        for v in range(_RPW1 // 16):
            ev = ebuf[pl.ds(v * 16, 16)]
            rank = jnp.zeros((16,), jnp.int32)
            for e in range(E):
                m = ev == e
                mi = jnp.where(m, jnp.int32(1), jnp.int32(0))
                cs = plsc.cumsum(mi)
                rank = jnp.where(m, carries[e] + cs - 1, rank)
                carries[e] = carries[e] + jnp.sum(mi)
            rankb[pl.ds(v * 16, 16)] = rank
        cvec = jnp.zeros((16,), jnp.int32)
        for e in range(E):
            cvec = jnp.where(lane == e, carries[e], cvec)
        cntv[...] = cvec
        pltpu.sync_copy(cntv, cnt_sh.at[sid])
        pltpu.sync_copy(psbuf, ps_sh.at[sid])
        pltpu.sync_copy(dnbuf, dn_sh.at[sid])

    plsc.subcore_barrier()

    @pl.when(active)
    def _phase2():
        tok0 = sid * _TPW1
        pltpu.sync_copy(cnt_sh, tabv)
        totals = jnp.zeros((16,), jnp.int32)
        myprefix = jnp.zeros((16,), jnp.int32)
        sidv = jnp.broadcast_to(sid, (16,))
        for w in range(_NW1):
            row = tabv[w]
            totals = totals + row
            before = jnp.full((16,), w, jnp.int32) < sidv
            myprefix = myprefix + jnp.where(before, row, 0)
        excl = plsc.cumsum(totals) - totals
        base = excl + myprefix
        for v in range(_RPW1 // 16):
            sl = pl.ds(v * 16, 16)
            ev = ebuf[sl]
            dst = rankb[sl]
            for e in range(E):
                dst = jnp.where(ev == e, dst + base[e], dst)
            invb[sl] = dst
        pltpu.sync_copy(invb.at[pl.ds(0, _TPW1)],
                        inv0_hbm.at[pl.ds(tok0, _TPW1)])
        pltpu.sync_copy(invb.at[pl.ds(_TPW1, _TPW1)],
                        inv1_hbm.at[pl.ds(tok0, _TPW1)])
        pltpu.sync_copy(g1b, g1_hbm.at[pl.ds(tok0, _TPW1)])
        pltpu.sync_copy(g2b, g2_hbm.at[pl.ds(tok0, _TPW1)])

        @pl.when(sid == 0)
        def _finalize():
            pltpu.sync_copy(ps_sh, pstab)
            pltpu.sync_copy(dn_sh, dntab)
            aux = jnp.float32(0.0)
            for e in range(E):
                pse = jnp.zeros((16,), jnp.float32)
                dne = jnp.zeros((16,), jnp.float32)
                for w in range(_NW1):
                    pse = pse + pstab[w, e]
                    dne = dne + dntab[w, e]
                aux = aux + jnp.sum(pse) * jnp.sum(dne)
            aux = aux * (COEF * E / (T * T))
            auxv[...] = jnp.where(lane == 0, aux, 0.0)
            pltpu.sync_copy(auxv, aux_hbm)

            s_g = [excl[g] for g in range(E)]
            t_g = [excl[g + 1] for g in range(E)]
            bs_g = [s_g[g] // B for g in range(E)]
            be_g = [jnp.where(t_g[g] > s_g[g], (t_g[g] + B - 1) // B,
                              bs_g[g]) for g in range(E)]
            st_g = [jnp.int32(0)]
            for g in range(E):
                st_g.append(st_g[g] + (be_g[g] - bs_g[g]))
            total_t = st_g[E]

            def tile_of(jv):
                g = jnp.zeros((16,), jnp.int32)
                for gg in range(1, E):
                    g = g + jnp.where(jv >= st_g[gg], 1, 0)
                bv = jnp.zeros((16,), jnp.int32)
                lov = jnp.zeros((16,), jnp.int32)
                hiv = jnp.zeros((16,), jnp.int32)
                for gg in range(E):
                    m = g == gg
                    b_gg = bs_g[gg] + jv - st_g[gg]
                    bv = jnp.where(m, b_gg, bv)
                    lov = jnp.where(m, jnp.maximum(s_g[gg], b_gg * B), lov)
                    hiv = jnp.where(m, jnp.minimum(t_g[gg], b_gg * B + B),
                                    hiv)
                return g, bv, lov, hiv

            for v in range(TM // 16):
                jv = jnp.int32(16 * v) + lane
                g, bv, lov, hiv = tile_of(jv)
                gp, bp, _, _ = tile_of(jv - 1)
                valid = jv < total_t
                zero = jnp.zeros((16,), jnp.int32)
                one = zero + 1
                vsl = pl.ds(16 * v, 16)
                tmbuf[0, vsl] = jnp.where(valid, g, jnp.int32(E - 1))
                tmbuf[1, vsl] = jnp.where(valid, bv, jnp.int32(NB - 1))
                tmbuf[2, vsl] = jnp.where(valid, lov, zero)
                tmbuf[3, vsl] = jnp.where(valid, hiv, zero)
                tmbuf[4, vsl] = jnp.where(valid & (bv != bp), one, zero)
                tmbuf[5, vsl] = jnp.where(
                    valid & ((jv == 0) | (g != gp)), one, zero)
            pltpu.sync_copy(tmbuf.at[0], grp_hbm)
            pltpu.sync_copy(tmbuf.at[1], blk_hbm)
            pltpu.sync_copy(tmbuf.at[2], lo_hbm)
            pltpu.sync_copy(tmbuf.at[3], hi_hbm)
            pltpu.sync_copy(tmbuf.at[4], first_hbm)
            pltpu.sync_copy(tmbuf.at[5], wchg_hbm)


def _dispatch(logits_flat):
    f = functools.partial(
        pl.kernel,
        out_type=(
            jax.ShapeDtypeStruct((T,), jnp.int32),
            jax.ShapeDtypeStruct((T,), jnp.int32),
            jax.ShapeDtypeStruct((T,), jnp.float32),
            jax.ShapeDtypeStruct((T,), jnp.float32),
            jax.ShapeDtypeStruct((16,), jnp.float32),
            jax.ShapeDtypeStruct((TM,), jnp.int32),
            jax.ShapeDtypeStruct((TM,), jnp.int32),
            jax.ShapeDtypeStruct((TM,), jnp.int32),
            jax.ShapeDtypeStruct((TM,), jnp.int32),
            jax.ShapeDtypeStruct((TM,), jnp.int32),
            jax.ShapeDtypeStruct((TM,), jnp.int32),
        ),
        mesh=plsc.VectorSubcoreMesh(core_axis_name="c", subcore_axis_name="s"),
        compiler_params=pltpu.CompilerParams(needs_layout_passes=False,
                                             use_tc_tiling_on_sc=False),
        scratch_types=[
            pltpu.VMEM((E, _TPW1), jnp.float32),
            pltpu.VMEM((_RPW1,), jnp.int32),
            pltpu.VMEM((_RPW1,), jnp.int32),
            pltpu.VMEM((_RPW1,), jnp.int32),
            pltpu.VMEM((_TPW1,), jnp.float32),
            pltpu.VMEM((_TPW1,), jnp.float32),
            pltpu.VMEM((16,), jnp.int32),
            pltpu.VMEM((E, 16), jnp.float32),
            pltpu.VMEM((E, 16), jnp.float32),
            pltpu.VMEM((_NW1, 16), jnp.int32),
            pltpu.VMEM((_NW1, E, 16), jnp.float32),
            pltpu.VMEM((_NW1, E, 16), jnp.float32),
            pltpu.VMEM((16,), jnp.float32),
            pltpu.VMEM((6, TM), jnp.int32),
            pltpu.VMEM_SHARED((_NW1, 16), jnp.int32),
            pltpu.VMEM_SHARED((_NW1, E, 16), jnp.float32),
            pltpu.VMEM_SHARED((_NW1, E, 16), jnp.float32),
            pltpu.SemaphoreType.DMA,
        ],
    )(_dispatch_body)
    return f(logits_flat)


def _xscatter_body(x_hbm, inv0_hbm, inv1_hbm, xs_hbm, xbuf, d0, d1, sem):
    wid = lax.axis_index("s") * 2 + lax.axis_index("c")
    tok0 = wid * _TPW
    for c in range(4):
        pltpu.sync_copy(inv0_hbm.at[pl.ds(tok0 + 32 * c, 32)], d0.at[c])
        pltpu.sync_copy(inv1_hbm.at[pl.ds(tok0 + 32 * c, 32)], d1.at[c])
    for c in range(4):
        pltpu.sync_copy(x_hbm.at[pl.ds(tok0 + 32 * c, 32)], xbuf)
        a = pltpu.async_copy(xbuf, xs_hbm.at[d0.at[c]], sem)
        b = pltpu.async_copy(xbuf, xs_hbm.at[d1.at[c]], sem)
        a.wait()
        b.wait()


def _xscatter(x, inv0, inv1):
    f = functools.partial(
        pl.kernel,
        out_type=jax.ShapeDtypeStruct((T * K, D), jnp.float32),
        mesh=plsc.VectorSubcoreMesh(core_axis_name="c", subcore_axis_name="s"),
        compiler_params=pltpu.CompilerParams(needs_layout_passes=False),
        scratch_types=[
            pltpu.VMEM((32, D), jnp.float32),
            pltpu.VMEM((4, 32), jnp.int32),
            pltpu.VMEM((4, 32), jnp.int32),
            pltpu.SemaphoreType.DMA,
        ],
    )(_xscatter_body)
    return f(x, inv0, inv1)


_NW = 32
_TPW = T // _NW
_CH = 16
_NCH = _TPW // _CH


def _combine_body(os_hbm, sh_hbm, inv0_hbm, inv1_hbm, g1_hbm, g2_hbm,
                  out_hbm,
                  i0b, i1b, rows0, rows1, sbuf, obuf, g1b, g2b, sem):
    wid = lax.axis_index("s") * 2 + lax.axis_index("c")
    tok0 = wid * _TPW
    pltpu.sync_copy(g1_hbm.at[pl.ds(tok0, _TPW)], g1b)
    pltpu.sync_copy(g2_hbm.at[pl.ds(tok0, _TPW)], g2b)
    for c in range(_NCH):
        pltpu.sync_copy(inv0_hbm.at[pl.ds(tok0 + c * _CH, _CH)], i0b.at[c])
        pltpu.sync_copy(inv1_hbm.at[pl.ds(tok0 + c * _CH, _CH)], i1b.at[c])
    for c in range(_NCH):
        a = pltpu.async_copy(os_hbm.at[i0b.at[c]], rows0, sem)
        b = pltpu.async_copy(os_hbm.at[i1b.at[c]], rows1, sem)
        pltpu.sync_copy(sh_hbm.at[pl.ds(tok0 + c * _CH, _CH)], sbuf)
        a.wait()
        b.wait()
        g1v = g1b[pl.ds(c * _CH, 16)]
        g2v = g2b[pl.ds(c * _CH, 16)]
        for j in range(_CH):
            ga = g1v[j]
            gb = g2v[j]

            def body(v, carry, j=j, ga=ga, gb=gb):
                sl = pl.ds(v * 16, 16)
                obuf[j, sl] = (ga * rows0[j, sl] + gb * rows1[j, sl]
                               + sbuf[j, sl])
                return carry

            lax.fori_loop(0, D // 16, body, 0)
        pltpu.sync_copy(obuf, out_hbm.at[pl.ds(tok0 + c * _CH, _CH)])


def _combine(os, shared, inv0, inv1, g1, g2):
    f = functools.partial(
        pl.kernel,
        out_type=jax.ShapeDtypeStruct((T, D), jnp.float32),
        mesh=plsc.VectorSubcoreMesh(core_axis_name="c", subcore_axis_name="s"),
        compiler_params=pltpu.CompilerParams(needs_layout_passes=False),
        scratch_types=[
            pltpu.VMEM((_NCH, _CH), jnp.int32),
            pltpu.VMEM((_NCH, _CH), jnp.int32),
            pltpu.VMEM((_CH, D), jnp.float32),
            pltpu.VMEM((_CH, D), jnp.float32),
            pltpu.VMEM((_CH, D), jnp.float32),
            pltpu.VMEM((_CH, D), jnp.float32),
            pltpu.VMEM((_TPW,), jnp.float32),
            pltpu.VMEM((_TPW,), jnp.float32),
            pltpu.SemaphoreType.DMA,
        ],
    )(_combine_body)
    return f(os, shared, inv0, inv1, g1, g2)


@jax.jit
def kernel(x, Wr, w1, w2, w3, sw1, sw2, sw3):
    logits = _router(x, Wr)

    (inv0, inv1, g1, g2, aux16,
     grp, blk, lo, hi, first, wchg) = _dispatch(logits.reshape(-1))
    aux_loss = aux16[0]

    shared = _shared_ffn(x, sw1.astype(jnp.bfloat16),
                         sw3.astype(jnp.bfloat16),
                         sw2.astype(jnp.bfloat16))

    xs = _xscatter(x, inv0, inv1)

    os = _grouped_ffn(xs, w1, w3, w2, grp, blk, lo, hi, first, wchg)

    out = _combine(os, shared, inv0, inv1, g1, g2)
    return out, aux_loss


# speedup vs baseline: 8.0088x; 1.1487x over previous
import functools
import jax
import jax.numpy as jnp
from jax import lax
from jax.experimental import pallas as pl
from jax.experimental.pallas import tpu as pltpu
from jax.experimental.pallas import tpu_sc as plsc

T = 4096
D = 1024
H = 1024
E = 8
K = 2
COEF = 0.01

B = 256
R = T * K
NB = R // B
NT = NB + E - 1
TM = 48


def _gelu(v):
    return 0.5 * v * (1.0 + jax.lax.erf(v * 0.7071067811865476))


def _router_body(x_ref, wr_ref, out_ref):
    out_ref[...] = jax.lax.dot_general(
        wr_ref[...], x_ref[...], (((1,), (1,)), ((), ())),
        preferred_element_type=jnp.float32)


def _router(x, Wr):
    blk = 512
    return pl.pallas_call(
        _router_body,
        grid=(T // blk,),
        in_specs=[
            pl.BlockSpec((blk, D), lambda i: (i, 0)),
            pl.BlockSpec((E, D), lambda i: (0, 0)),
        ],
        out_specs=pl.BlockSpec((E, blk), lambda i: (0, i)),
        out_shape=jax.ShapeDtypeStruct((E, T), jnp.float32),
    )(x, Wr)


def _ffn_body(grp_ref, blk_ref, lo_ref, hi_ref, first_ref, wchg_ref,
              xs_ref, w1_ref, w3_ref, w2_ref, out_ref, ws1, ws3, ws2):
    i = pl.program_id(0)
    lo = lo_ref[i]
    hi = hi_ref[i]
    base = blk_ref[i] * B
    rows = base + jax.lax.broadcasted_iota(jnp.int32, (B, 1), 0)
    mask = (rows >= lo) & (rows < hi)

    @pl.when(wchg_ref[i] == 1)
    def _():
        ws1[...] = w1_ref[0].astype(jnp.bfloat16)
        ws3[...] = w3_ref[0].astype(jnp.bfloat16)
        ws2[...] = w2_ref[0].astype(jnp.bfloat16)

    x = xs_ref[...].astype(jnp.bfloat16)
    h1 = jax.lax.dot_general(x, ws1[...], (((1,), (1,)), ((), ())),
                             preferred_element_type=jnp.float32)
    h3 = jax.lax.dot_general(x, ws3[...], (((1,), (1,)), ((), ())),
                             preferred_element_type=jnp.float32)
    h = (_gelu(h1) * h3).astype(jnp.bfloat16)
    o = jax.lax.dot_general(h, ws2[...], (((1,), (1,)), ((), ())),
                            preferred_element_type=jnp.float32)
    o = jnp.where(mask, o, 0.0)

    @pl.when(first_ref[i] == 1)
    def _():
        out_ref[...] = o

    @pl.when(first_ref[i] == 0)
    def _():
        out_ref[...] += o


def _grouped_ffn(xs, w1, w3, w2, grp, blk, lo, hi, first, wchg):
    grid_spec = pltpu.PrefetchScalarGridSpec(
        num_scalar_prefetch=6,
        grid=(NT,),
        in_specs=[
            pl.BlockSpec((B, D), lambda i, g, b, lo, hi, f, w: (b[i], 0)),
            pl.BlockSpec((1, H, D), lambda i, g, b, lo, hi, f, w: (g[i], 0, 0)),
            pl.BlockSpec((1, H, D), lambda i, g, b, lo, hi, f, w: (g[i], 0, 0)),
            pl.BlockSpec((1, D, H), lambda i, g, b, lo, hi, f, w: (g[i], 0, 0)),
        ],
        out_specs=pl.BlockSpec((B, D), lambda i, g, b, lo, hi, f, w: (b[i], 0)),
        scratch_shapes=[
            pltpu.VMEM((H, D), jnp.bfloat16),
            pltpu.VMEM((H, D), jnp.bfloat16),
            pltpu.VMEM((D, H), jnp.bfloat16),
        ],
    )
    return pl.pallas_call(
        _ffn_body,
        grid_spec=grid_spec,
        out_shape=jax.ShapeDtypeStruct((R, D), jnp.float32),
    )(grp, blk, lo, hi, first, wchg, xs, w1, w3, w2)


def _sffn_body(x_ref, w1_ref, w3_ref, w2_ref, out_ref, ws1, ws3, ws2):
    @pl.when(pl.program_id(0) == 0)
    def _():
        ws1[...] = w1_ref[...].astype(jnp.bfloat16)
        ws3[...] = w3_ref[...].astype(jnp.bfloat16)
        ws2[...] = w2_ref[...].astype(jnp.bfloat16)

    x = x_ref[...].astype(jnp.bfloat16)
    h1 = jax.lax.dot_general(x, ws1[...], (((1,), (1,)), ((), ())),
                             preferred_element_type=jnp.float32)
    h3 = jax.lax.dot_general(x, ws3[...], (((1,), (1,)), ((), ())),
                             preferred_element_type=jnp.float32)
    h = (_gelu(h1) * h3).astype(jnp.bfloat16)
    out_ref[...] = jax.lax.dot_general(h, ws2[...], (((1,), (1,)), ((), ())),
                                       preferred_element_type=jnp.float32)


def _shared_ffn(x, sw1, sw3, sw2):
    blk = 512
    return pl.pallas_call(
        _sffn_body,
        grid=(T // blk,),
        in_specs=[
            pl.BlockSpec((blk, D), lambda i: (i, 0)),
            pl.BlockSpec((H, D), lambda i: (0, 0)),
            pl.BlockSpec((H, D), lambda i: (0, 0)),
            pl.BlockSpec((D, H), lambda i: (0, 0)),
        ],
        out_specs=pl.BlockSpec((blk, D), lambda i: (i, 0)),
        out_shape=jax.ShapeDtypeStruct((T, D), jnp.float32),
        scratch_shapes=[
            pltpu.VMEM((H, D), jnp.bfloat16),
            pltpu.VMEM((H, D), jnp.bfloat16),
            pltpu.VMEM((D, H), jnp.bfloat16),
        ],
    )(x, sw1, sw3, sw2)


_NW1 = 16
_TPW1 = T // _NW1
_RPW1 = _TPW1 * K
_NCH1 = _TPW1 // 16


def _dispatch_body(lg_hbm, inv0_hbm, inv1_hbm, g1_hbm, g2_hbm,
                   aux_hbm, grp_hbm, blk_hbm, lo_hbm, hi_hbm,
                   first_hbm, wchg_hbm,
                   lbuf, ebuf, invb, rankb, g1b, g2b, cntv,
                   psbuf, dnbuf, tabv, pstab, dntab, auxv, tmbuf,
                   cnt_sh, ps_sh, dn_sh, sem):
    cid = lax.axis_index("c")
    sid = lax.axis_index("s")
    active = cid == 0
    lane = lax.iota(jnp.int32, 16)

    @pl.when(active)
    def _phase1():
        tok0 = sid * _TPW1
        for e in range(E):
            pltpu.sync_copy(lg_hbm.at[pl.ds(e * T + tok0, _TPW1)],
                            lbuf.at[e])
        ps_acc = [jnp.zeros((16,), jnp.float32) for _ in range(E)]
        dn_acc = [jnp.zeros((16,), jnp.float32) for _ in range(E)]
        for j in range(_NCH1):
            sl = pl.ds(j * 16, 16)
            le = [lbuf[e, sl] for e in range(E)]
            m1 = le[0]
            i1 = jnp.zeros((16,), jnp.int32)
            for e in range(1, E):
                gt = le[e] > m1
                i1 = jnp.where(gt, jnp.int32(e), i1)
                m1 = jnp.where(gt, le[e], m1)
            m2 = jnp.full((16,), -3e38, jnp.float32)
            i2 = jnp.zeros((16,), jnp.int32)
            for e in range(E):
                gt = (le[e] > m2) & (i1 != e)
                i2 = jnp.where(gt, jnp.int32(e), i2)
                m2 = jnp.where(gt, le[e], m2)
            ex = [jnp.exp(le[e] - m1) for e in range(E)]
            sumexp = ex[0]
            for e in range(1, E):
                sumexp = sumexp + ex[e]
            g1 = 1.0 / sumexp
            g2 = jnp.exp(m2 - m1) * g1
            g1b[sl] = g1
            g2b[sl] = g2
            for e in range(E):
                ps_acc[e] = ps_acc[e] + ex[e] * g1
                dn_acc[e] = dn_acc[e] + jnp.where(i1 == e, 1.0, 0.0)
            ebuf[sl] = i1
            ebuf[pl.ds(_TPW1 + j * 16, 16)] = i2
        for e in range(E):
            psbuf[e] = ps_acc[e]
            dnbuf[e] = dn_acc[e]
        carries = [jnp.int32(0) for _ in range(E)]
        for v in range(_RPW1 // 16):
            ev = ebuf[pl.ds(v * 16, 16)]
            rank = jnp.zeros((16,), jnp.int32)
            for e in range(E):
                m = ev == e
                mi = jnp.where(m, jnp.int32(1), jnp.int32(0))
                cs = plsc.cumsum(mi)
                rank = jnp.where(m, carries[e] + cs - 1, rank)
                carries[e] = carries[e] + jnp.sum(mi)
            rankb[pl.ds(v * 16, 16)] = rank
        cvec = jnp.zeros((16,), jnp.int32)
        for e in range(E):
            cvec = jnp.where(lane == e, carries[e], cvec)
        cntv[...] = cvec
        pltpu.sync_copy(cntv, cnt_sh.at[sid])
        pltpu.sync_copy(psbuf, ps_sh.at[sid])
        pltpu.sync_copy(dnbuf, dn_sh.at[sid])

    plsc.subcore_barrier()

    @pl.when(active)
    def _phase2():
        tok0 = sid * _TPW1
        pltpu.sync_copy(cnt_sh, tabv)
        totals = jnp.zeros((16,), jnp.int32)
        myprefix = jnp.zeros((16,), jnp.int32)
        sidv = jnp.broadcast_to(sid, (16,))
        for w in range(_NW1):
            row = tabv[w]
            totals = totals + row
            before = jnp.full((16,), w, jnp.int32) < sidv
            myprefix = myprefix + jnp.where(before, row, 0)
        excl = plsc.cumsum(totals) - totals
        base = excl + myprefix
        for v in range(_RPW1 // 16):
            sl = pl.ds(v * 16, 16)
            ev = ebuf[sl]
            dst = rankb[sl]
            for e in range(E):
                dst = jnp.where(ev == e, dst + base[e], dst)
            invb[sl] = dst
        pltpu.sync_copy(invb.at[pl.ds(0, _TPW1)],
                        inv0_hbm.at[pl.ds(tok0, _TPW1)])
        pltpu.sync_copy(invb.at[pl.ds(_TPW1, _TPW1)],
                        inv1_hbm.at[pl.ds(tok0, _TPW1)])
        pltpu.sync_copy(g1b, g1_hbm.at[pl.ds(tok0, _TPW1)])
        pltpu.sync_copy(g2b, g2_hbm.at[pl.ds(tok0, _TPW1)])

        @pl.when(sid == 0)
        def _finalize():
            pltpu.sync_copy(ps_sh, pstab)
            pltpu.sync_copy(dn_sh, dntab)
            aux = jnp.float32(0.0)
            for e in range(E):
                pse = jnp.zeros((16,), jnp.float32)
                dne = jnp.zeros((16,), jnp.float32)
                for w in range(_NW1):
                    pse = pse + pstab[w, e]
                    dne = dne + dntab[w, e]
                aux = aux + jnp.sum(pse) * jnp.sum(dne)
            aux = aux * (COEF * E / (T * T))
            auxv[...] = jnp.where(lane == 0, aux, 0.0)
            pltpu.sync_copy(auxv, aux_hbm)

            s_g = [excl[g] for g in range(E)]
            t_g = [excl[g + 1] for g in range(E)]
            bs_g = [s_g[g] // B for g in range(E)]
            be_g = [jnp.where(t_g[g] > s_g[g], (t_g[g] + B - 1) // B,
                              bs_g[g]) for g in range(E)]
            st_g = [jnp.int32(0)]
            for g in range(E):
                st_g.append(st_g[g] + (be_g[g] - bs_g[g]))
            total_t = st_g[E]

            def tile_of(jv):
                g = jnp.zeros((16,), jnp.int32)
                for gg in range(1, E):
                    g = g + jnp.where(jv >= st_g[gg], 1, 0)
                bv = jnp.zeros((16,), jnp.int32)
                lov = jnp.zeros((16,), jnp.int32)
                hiv = jnp.zeros((16,), jnp.int32)
                for gg in range(E):
                    m = g == gg
                    b_gg = bs_g[gg] + jv - st_g[gg]
                    bv = jnp.where(m, b_gg, bv)
                    lov = jnp.where(m, jnp.maximum(s_g[gg], b_gg * B), lov)
                    hiv = jnp.where(m, jnp.minimum(t_g[gg], b_gg * B + B),
                                    hiv)
                return g, bv, lov, hiv

            for v in range(TM // 16):
                jv = jnp.int32(16 * v) + lane
                g, bv, lov, hiv = tile_of(jv)
                gp, bp, _, _ = tile_of(jv - 1)
                valid = jv < total_t
                zero = jnp.zeros((16,), jnp.int32)
                one = zero + 1
                vsl = pl.ds(16 * v, 16)
                tmbuf[0, vsl] = jnp.where(valid, g, jnp.int32(E - 1))
                tmbuf[1, vsl] = jnp.where(valid, bv, jnp.int32(NB - 1))
                tmbuf[2, vsl] = jnp.where(valid, lov, zero)
                tmbuf[3, vsl] = jnp.where(valid, hiv, zero)
                tmbuf[4, vsl] = jnp.where(valid & (bv != bp), one, zero)
                tmbuf[5, vsl] = jnp.where(
                    valid & ((jv == 0) | (g != gp)), one, zero)
            pltpu.sync_copy(tmbuf.at[0], grp_hbm)
            pltpu.sync_copy(tmbuf.at[1], blk_hbm)
            pltpu.sync_copy(tmbuf.at[2], lo_hbm)
            pltpu.sync_copy(tmbuf.at[3], hi_hbm)
            pltpu.sync_copy(tmbuf.at[4], first_hbm)
            pltpu.sync_copy(tmbuf.at[5], wchg_hbm)


def _dispatch(logits_flat):
    f = functools.partial(
        pl.kernel,
        out_type=(
            jax.ShapeDtypeStruct((T,), jnp.int32),
            jax.ShapeDtypeStruct((T,), jnp.int32),
            jax.ShapeDtypeStruct((T,), jnp.float32),
            jax.ShapeDtypeStruct((T,), jnp.float32),
            jax.ShapeDtypeStruct((16,), jnp.float32),
            jax.ShapeDtypeStruct((TM,), jnp.int32),
            jax.ShapeDtypeStruct((TM,), jnp.int32),
            jax.ShapeDtypeStruct((TM,), jnp.int32),
            jax.ShapeDtypeStruct((TM,), jnp.int32),
            jax.ShapeDtypeStruct((TM,), jnp.int32),
            jax.ShapeDtypeStruct((TM,), jnp.int32),
        ),
        mesh=plsc.VectorSubcoreMesh(core_axis_name="c", subcore_axis_name="s"),
        compiler_params=pltpu.CompilerParams(needs_layout_passes=False,
                                             use_tc_tiling_on_sc=False),
        scratch_types=[
            pltpu.VMEM((E, _TPW1), jnp.float32),
            pltpu.VMEM((_RPW1,), jnp.int32),
            pltpu.VMEM((_RPW1,), jnp.int32),
            pltpu.VMEM((_RPW1,), jnp.int32),
            pltpu.VMEM((_TPW1,), jnp.float32),
            pltpu.VMEM((_TPW1,), jnp.float32),
            pltpu.VMEM((16,), jnp.int32),
            pltpu.VMEM((E, 16), jnp.float32),
            pltpu.VMEM((E, 16), jnp.float32),
            pltpu.VMEM((_NW1, 16), jnp.int32),
            pltpu.VMEM((_NW1, E, 16), jnp.float32),
            pltpu.VMEM((_NW1, E, 16), jnp.float32),
            pltpu.VMEM((16,), jnp.float32),
            pltpu.VMEM((6, TM), jnp.int32),
            pltpu.VMEM_SHARED((_NW1, 16), jnp.int32),
            pltpu.VMEM_SHARED((_NW1, E, 16), jnp.float32),
            pltpu.VMEM_SHARED((_NW1, E, 16), jnp.float32),
            pltpu.SemaphoreType.DMA,
        ],
    )(_dispatch_body)
    return f(logits_flat)


def _xscatter_body(x_hbm, inv0_hbm, inv1_hbm, xs_hbm, xbuf, d0, d1, sem):
    wid = lax.axis_index("s") * 2 + lax.axis_index("c")
    tok0 = wid * _TPW
    for c in range(4):
        pltpu.sync_copy(inv0_hbm.at[pl.ds(tok0 + 32 * c, 32)], d0.at[c])
        pltpu.sync_copy(inv1_hbm.at[pl.ds(tok0 + 32 * c, 32)], d1.at[c])
    for c in range(4):
        pltpu.sync_copy(x_hbm.at[pl.ds(tok0 + 32 * c, 32)], xbuf)
        a = pltpu.async_copy(xbuf, xs_hbm.at[d0.at[c]], sem)
        b = pltpu.async_copy(xbuf, xs_hbm.at[d1.at[c]], sem)
        a.wait()
        b.wait()


def _xscatter(x, inv0, inv1):
    f = functools.partial(
        pl.kernel,
        out_type=jax.ShapeDtypeStruct((T * K, D), jnp.float32),
        mesh=plsc.VectorSubcoreMesh(core_axis_name="c", subcore_axis_name="s"),
        compiler_params=pltpu.CompilerParams(needs_layout_passes=False),
        scratch_types=[
            pltpu.VMEM((32, D), jnp.float32),
            pltpu.VMEM((4, 32), jnp.int32),
            pltpu.VMEM((4, 32), jnp.int32),
            pltpu.SemaphoreType.DMA,
        ],
    )(_xscatter_body)
    return f(x, inv0, inv1)


_NW = 32
_TPW = T // _NW
_CH = 16
_NCH = _TPW // _CH


def _combine_body(os_hbm, sh_hbm, inv0_hbm, inv1_hbm, g1_hbm, g2_hbm,
                  out_hbm,
                  i0b, i1b, rows0, rows1, sbuf, obuf, g1b, g2b,
                  sem0, sem1):
    wid = lax.axis_index("s") * 2 + lax.axis_index("c")
    tok0 = wid * _TPW
    sems = (sem0, sem1)
    pltpu.sync_copy(g1_hbm.at[pl.ds(tok0, _TPW)], g1b)
    pltpu.sync_copy(g2_hbm.at[pl.ds(tok0, _TPW)], g2b)
    for c in range(_NCH):
        pltpu.sync_copy(inv0_hbm.at[pl.ds(tok0 + c * _CH, _CH)], i0b.at[c])
        pltpu.sync_copy(inv1_hbm.at[pl.ds(tok0 + c * _CH, _CH)], i1b.at[c])

    def issue(c, p):
        return (
            pltpu.async_copy(os_hbm.at[i0b.at[c]], rows0.at[p], sems[p]),
            pltpu.async_copy(os_hbm.at[i1b.at[c]], rows1.at[p], sems[p]),
            pltpu.async_copy(sh_hbm.at[pl.ds(tok0 + c * _CH, _CH)],
                             sbuf.at[p], sems[p]),
        )

    pending = {0: issue(0, 0)}
    for c in range(_NCH):
        p = c % 2
        if c + 1 < _NCH:
            pending[1 - p] = issue(c + 1, 1 - p)
        for h in pending[p]:
            h.wait()
        g1v = g1b[pl.ds(c * _CH, 16)]
        g2v = g2b[pl.ds(c * _CH, 16)]
        for j in range(_CH):
            ga = g1v[j]
            gb = g2v[j]

            def body(v, carry, j=j, ga=ga, gb=gb, p=p):
                sl = pl.ds(v * 16, 16)
                obuf[j, sl] = (ga * rows0[p, j, sl] + gb * rows1[p, j, sl]
                               + sbuf[p, j, sl])
                return carry

            lax.fori_loop(0, D // 16, body, 0)
        pltpu.sync_copy(obuf, out_hbm.at[pl.ds(tok0 + c * _CH, _CH)])


def _combine(os, shared, inv0, inv1, g1, g2):
    f = functools.partial(
        pl.kernel,
        out_type=jax.ShapeDtypeStruct((T, D), jnp.float32),
        mesh=plsc.VectorSubcoreMesh(core_axis_name="c", subcore_axis_name="s"),
        compiler_params=pltpu.CompilerParams(needs_layout_passes=False),
        scratch_types=[
            pltpu.VMEM((_NCH, _CH), jnp.int32),
            pltpu.VMEM((_NCH, _CH), jnp.int32),
            pltpu.VMEM((2, _CH, D), jnp.float32),
            pltpu.VMEM((2, _CH, D), jnp.float32),
            pltpu.VMEM((2, _CH, D), jnp.float32),
            pltpu.VMEM((_CH, D), jnp.float32),
            pltpu.VMEM((_TPW,), jnp.float32),
            pltpu.VMEM((_TPW,), jnp.float32),
            pltpu.SemaphoreType.DMA,
            pltpu.SemaphoreType.DMA,
        ],
    )(_combine_body)
    return f(os, shared, inv0, inv1, g1, g2)


@jax.jit
def kernel(x, Wr, w1, w2, w3, sw1, sw2, sw3):
    logits = _router(x, Wr)

    (inv0, inv1, g1, g2, aux16,
     grp, blk, lo, hi, first, wchg) = _dispatch(logits.reshape(-1))
    aux_loss = aux16[0]

    shared = _shared_ffn(x, sw1, sw3, sw2)

    xs = _xscatter(x, inv0, inv1)

    os = _grouped_ffn(xs, w1, w3, w2, grp, blk, lo, hi, first, wchg)

    out = _combine(os, shared, inv0, inv1, g1, g2)
    return out, aux_loss
